# baseline (speedup 1.0000x reference)
import functools

import jax
import jax.numpy as jnp
from jax import lax
from jax.experimental import pallas as pl
from jax.experimental.pallas import tpu as pltpu
from jax.experimental.pallas import tpu_sc as plsc


_L = 16
_BW = 128


@functools.lru_cache(maxsize=None)
def _make_gather(num_rows: int, feat: int, nb: int, nh: int):
    info = plsc.get_sparse_core_info()
    nc, ns = info.num_cores, info.num_subcores
    nw = nc * ns
    assert nb == nw * _BW and feat == 32 and nh % 2 == 0
    ftiles = feat // 8
    mesh = plsc.VectorSubcoreMesh(core_axis_name="core", subcore_axis_name="sub")

    @functools.partial(
        pl.kernel,
        out_type=jax.ShapeDtypeStruct((nh * ftiles * nw * 8, _BW), jnp.float32),
        mesh=mesh,
        scratch_types=[
            pltpu.VMEM((nh, _BW), jnp.int32),
            [pltpu.VMEM((_BW, feat), jnp.float32)] * 2,
            [pltpu.VMEM((feat, _BW), jnp.float32)] * 2,
            [pltpu.SemaphoreType.DMA] * 2,
            [pltpu.SemaphoreType.DMA] * 2,
        ],
        compiler_params=pltpu.CompilerParams(
            use_tc_tiling_on_sc=False, needs_layout_passes=False),
    )
    def gather_kernel(idx_hbm, table_hbm, out_hbm, idx_v, rows_v, t_v,
                      sems_g, sems_s):
        w = lax.axis_index("sub") * nc + lax.axis_index("core")
        pltpu.sync_copy(idx_hbm.at[:, pl.ds(w * _BW, _BW)], idx_v)

        iota = lax.iota(jnp.int32, _L)

        def fire(h, b):
            return pltpu.async_copy(
                table_hbm.at[idx_v.at[h]], rows_v[b], sems_g[b])

        def transpose(b):
            def tbody(f, carry):
                fv = jnp.full((_L,), f, dtype=jnp.int32)
                for k in range(_BW // _L):
                    col = plsc.load_gather(rows_v[b], [iota + (k * _L), fv])
                    t_v[b][f, pl.ds(k * _L, _L)] = col
                return carry
            lax.fori_loop(0, feat, tbody, 0)

        def store(h, b):
            return [
                pltpu.async_copy(
                    t_v[b].at[pl.ds(i * 8, 8)],
                    out_hbm.at[pl.ds((((h * ftiles) + i) * nw + w) * 8, 8)],
                    sems_s[b],
                )
                for i in range(ftiles)
            ]

        def body(p, carry):
            h0 = p * 2
            g0 = fire(h0, 0)
            g1 = fire(h0 + 1, 1)
            g0.wait()
            transpose(0)
            s0 = store(h0, 0)
            g1.wait()
            transpose(1)
            s1 = store(h0 + 1, 1)
            for cp in s0 + s1:
                cp.wait()
            return carry

        lax.fori_loop(0, nh // 2, body, 0)

    def run(x, table):
        idx_hm = jnp.transpose(x, (1, 2, 0)).reshape(nh, nb)
        out = gather_kernel(idx_hm, table)
        out5 = out.reshape(nh, ftiles, nw, 8, _BW)
        return jnp.transpose(out5, (2, 4, 0, 1, 3)).reshape(nb, nh, feat)

    return run


def kernel(x, table):
    b, h = x.shape[0], x.shape[1]
    run = _make_gather(table.shape[0], table.shape[1], b, h)
    return run(x, table)


# speedup vs baseline: 1.6628x; 1.6628x over previous
import functools

import jax
import jax.numpy as jnp
from jax import lax
from jax.experimental import pallas as pl
from jax.experimental.pallas import tpu as pltpu
from jax.experimental.pallas import tpu_sc as plsc


_L = 16
_BW = 128
_TP = 129


@functools.lru_cache(maxsize=None)
def _make_gather(num_rows: int, feat: int, nb: int, nh: int):
    info = plsc.get_sparse_core_info()
    nc, ns = info.num_cores, info.num_subcores
    nw = nc * ns
    assert nb == nw * _BW and feat == 32 and nh % 2 == 0
    ftiles = feat // 8
    mesh = plsc.VectorSubcoreMesh(core_axis_name="core", subcore_axis_name="sub")

    @functools.partial(
        pl.kernel,
        out_type=jax.ShapeDtypeStruct((nh * ftiles * nw * 8, _BW), jnp.float32),
        mesh=mesh,
        scratch_types=[
            pltpu.VMEM((nh, _BW), jnp.int32),
            [pltpu.VMEM((_BW, feat), jnp.float32)] * 2,
            [pltpu.VMEM((feat, _TP), jnp.float32)] * 2,
            [pltpu.SemaphoreType.DMA] * 2,
            [pltpu.SemaphoreType.DMA] * 2,
        ],
        compiler_params=pltpu.CompilerParams(
            use_tc_tiling_on_sc=False, needs_layout_passes=False),
    )
    def gather_kernel(idx_hbm, table_hbm, out_hbm, idx_v, rows_v, t_v,
                      sems_g, sems_s):
        w = lax.axis_index("sub") * nc + lax.axis_index("core")
        pltpu.sync_copy(idx_hbm.at[:, pl.ds(w * _BW, _BW)], idx_v)

        iota = lax.iota(jnp.int32, _L)
        f_lo = iota
        f_hi = iota + _L

        def fire(h, b):
            return pltpu.async_copy(
                table_hbm.at[idx_v.at[h]], rows_v[b], sems_g[b])

        def transpose(b):
            def tbody(l4, carry):
                for k in range(4):
                    l = l4 * 4 + k
                    lv = jnp.full((_L,), l, dtype=jnp.int32)
                    v1 = rows_v[b][l, pl.ds(0, _L)]
                    v2 = rows_v[b][l, pl.ds(_L, _L)]
                    plsc.store_scatter(t_v[b], [f_lo, lv], v1)
                    plsc.store_scatter(t_v[b], [f_hi, lv], v2)
                return carry
            lax.fori_loop(0, _BW // 4, tbody, 0)

        def store(h, b):
            return [
                pltpu.async_copy(
                    t_v[b].at[pl.ds(i * 8, 8), pl.ds(0, _BW)],
                    out_hbm.at[pl.ds((((h * ftiles) + i) * nw + w) * 8, 8)],
                    sems_s[b],
                )
                for i in range(ftiles)
            ]

        def body(p, carry):
            h0 = p * 2
            g0 = fire(h0, 0)
            g1 = fire(h0 + 1, 1)
            g0.wait()
            transpose(0)
            s0 = store(h0, 0)
            g1.wait()
            transpose(1)
            s1 = store(h0 + 1, 1)
            for cp in s0 + s1:
                cp.wait()
            return carry

        lax.fori_loop(0, nh // 2, body, 0)

    def run(x, table):
        idx_hm = jnp.transpose(x, (1, 2, 0)).reshape(nh, nb)
        out = gather_kernel(idx_hm, table)
        out5 = out.reshape(nh, ftiles, nw, 8, _BW)
        return jnp.transpose(out5, (2, 4, 0, 1, 3)).reshape(nb, nh, feat)

    return run


def kernel(x, table):
    b, h = x.shape[0], x.shape[1]
    run = _make_gather(table.shape[0], table.shape[1], b, h)
    return run(x, table)
